# baseline (speedup 1.0000x reference)
import functools

import jax
import jax.numpy as jnp
from jax import lax
from jax.experimental import pallas as pl
from jax.experimental.pallas import tpu as pltpu
from jax.experimental.pallas import tpu_sc as plsc

_TIMESTEPS = 1000
_B = 16384
_NC, _NS, _L = 1, 16, 16
_NW = _NC * _NS
_CHUNK = _B // _NW
_HALF = _CHUNK // 2
_NVEC = _CHUNK // _L
_TBL = _TIMESTEPS + 1

_mesh = plsc.VectorSubcoreMesh(
    core_axis_name="c", subcore_axis_name="s", num_cores=1
)


@functools.partial(
    pl.kernel,
    out_type=jax.ShapeDtypeStruct((_B,), jnp.float32),
    mesh=_mesh,
    scratch_types=[
        pltpu.VMEM((_CHUNK,), jnp.float32),
        pltpu.VMEM((_TBL,), jnp.float32),
        pltpu.VMEM((_CHUNK,), jnp.float32),
        pltpu.SemaphoreType.DMA,
        pltpu.SemaphoreType.DMA,
        pltpu.SemaphoreType.DMA,
        pltpu.SemaphoreType.DMA,
    ],
    compiler_params=pltpu.CompilerParams(
        needs_layout_passes=False,
        disable_bounds_checks=True,
        disable_semaphore_checks=True,
        skip_device_barrier=True,
    ),
)
def _betas_lookup(
    t_hbm, betas_hbm, out_hbm, t_v, tbl_v, out_v, sem_t, sem_b, sem_o0, sem_o1
):
    wid = lax.axis_index("s")
    base = wid * _CHUNK
    cp_t = pltpu.async_copy(t_hbm.at[pl.ds(base, _CHUNK)], t_v, sem_t)
    cp_b = pltpu.async_copy(betas_hbm, tbl_v, sem_b)
    cp_t.wait()
    cp_b.wait()

    def step(i):
        x = t_v[pl.ds(i, _L)] * jnp.float32(_TIMESTEPS)
        y = x + jnp.float32(0.5)
        idx = y.astype(jnp.int32)
        tie = idx.astype(jnp.float32) == y
        idx = idx - jnp.where(tie, idx & 1, 0)
        out_v[pl.ds(i, _L)] = plsc.load_gather(tbl_v, [idx])

    plsc.parallel_loop(0, _HALF, _L, unroll=8)(step)
    cp_o0 = pltpu.async_copy(
        out_v.at[pl.ds(0, _HALF)], out_hbm.at[pl.ds(base, _HALF)], sem_o0
    )
    plsc.parallel_loop(_HALF, _CHUNK, _L, unroll=8)(step)
    cp_o1 = pltpu.async_copy(
        out_v.at[pl.ds(_HALF, _HALF)], out_hbm.at[pl.ds(base + _HALF, _HALF)], sem_o1
    )
    cp_o0.wait()
    cp_o1.wait()


def kernel(t_normalized, betas):
    return _betas_lookup(t_normalized, betas)


# speedup vs baseline: 1.0090x; 1.0090x over previous
import functools

import jax
import jax.numpy as jnp
from jax import lax
from jax.experimental import pallas as pl
from jax.experimental.pallas import tpu as pltpu
from jax.experimental.pallas import tpu_sc as plsc

_TIMESTEPS = 1000
_B = 16384
_NC, _NS, _L = 1, 16, 16
_NW = _NC * _NS
_CHUNK = _B // _NW
_HALF = _CHUNK // 2
_NVEC = _CHUNK // _L
_TBL = _TIMESTEPS + 1

_mesh = plsc.VectorSubcoreMesh(
    core_axis_name="c", subcore_axis_name="s", num_cores=1
)


@functools.partial(
    pl.kernel,
    out_type=jax.ShapeDtypeStruct((_B,), jnp.float32),
    mesh=_mesh,
    scratch_types=[
        pltpu.VMEM((_CHUNK,), jnp.float32),
        pltpu.VMEM((_TBL,), jnp.float32),
        pltpu.VMEM((_CHUNK,), jnp.float32),
        pltpu.SemaphoreType.DMA,
        pltpu.SemaphoreType.DMA,
        pltpu.SemaphoreType.DMA,
        pltpu.SemaphoreType.DMA,
    ],
    compiler_params=pltpu.CompilerParams(
        needs_layout_passes=False,
        disable_bounds_checks=True,
        disable_semaphore_checks=True,
        skip_device_barrier=True,
    ),
)
def _betas_lookup(
    t_hbm, betas_hbm, out_hbm, t_v, tbl_v, out_v, sem_t, sem_b, sem_o0, sem_o1
):
    wid = lax.axis_index("s")
    base = wid * _CHUNK
    cp_t = pltpu.async_copy(t_hbm.at[pl.ds(base, _CHUNK)], t_v, sem_t)
    cp_b = pltpu.async_copy(betas_hbm, tbl_v, sem_b)
    cp_t.wait()
    cp_b.wait()

    def step(i):
        x = t_v[pl.ds(i, _L)] * jnp.float32(_TIMESTEPS)
        y = x + jnp.float32(0.5)
        idx = y.astype(jnp.int32)
        tie = idx.astype(jnp.float32) == y
        idx = idx - jnp.where(tie, idx & 1, 0)
        out_v[pl.ds(i, _L)] = plsc.load_gather(tbl_v, [idx])

    plsc.parallel_loop(0, _HALF, _L, unroll=4)(step)
    cp_o0 = pltpu.async_copy(
        out_v.at[pl.ds(0, _HALF)], out_hbm.at[pl.ds(base, _HALF)], sem_o0
    )
    plsc.parallel_loop(_HALF, _CHUNK, _L, unroll=4)(step)
    cp_o1 = pltpu.async_copy(
        out_v.at[pl.ds(_HALF, _HALF)], out_hbm.at[pl.ds(base + _HALF, _HALF)], sem_o1
    )
    cp_o0.wait()
    cp_o1.wait()


def kernel(t_normalized, betas):
    return _betas_lookup(t_normalized, betas)


# speedup vs baseline: 1.0095x; 1.0005x over previous
import functools

import jax
import jax.numpy as jnp
from jax import lax
from jax.experimental import pallas as pl
from jax.experimental.pallas import tpu as pltpu
from jax.experimental.pallas import tpu_sc as plsc

_TIMESTEPS = 1000
_B = 16384
_NC, _NS, _L = 1, 16, 16
_NW = _NC * _NS
_CHUNK = _B // _NW
_HALF = _CHUNK // 2
_NVEC = _CHUNK // _L
_TBL = _TIMESTEPS + 1

_mesh = plsc.VectorSubcoreMesh(
    core_axis_name="c", subcore_axis_name="s", num_cores=1
)


@functools.partial(
    pl.kernel,
    out_type=jax.ShapeDtypeStruct((_B,), jnp.float32),
    mesh=_mesh,
    scratch_types=[
        pltpu.VMEM((_CHUNK,), jnp.float32),
        pltpu.VMEM((_TBL,), jnp.float32),
        pltpu.VMEM((_CHUNK,), jnp.float32),
        pltpu.SemaphoreType.DMA,
        pltpu.SemaphoreType.DMA,
        pltpu.SemaphoreType.DMA,
        pltpu.SemaphoreType.DMA,
    ],
    compiler_params=pltpu.CompilerParams(needs_layout_passes=False),
)
def _betas_lookup(
    t_hbm, betas_hbm, out_hbm, t_v, tbl_v, out_v, sem_t, sem_b, sem_o0, sem_o1
):
    wid = lax.axis_index("s")
    base = wid * _CHUNK
    cp_t = pltpu.async_copy(t_hbm.at[pl.ds(base, _CHUNK)], t_v, sem_t)
    cp_b = pltpu.async_copy(betas_hbm, tbl_v, sem_b)
    cp_t.wait()
    cp_b.wait()

    def step(i):
        x = t_v[pl.ds(i, _L)] * jnp.float32(_TIMESTEPS)
        y = x + jnp.float32(0.5)
        idx = y.astype(jnp.int32)
        tie = idx.astype(jnp.float32) == y
        idx = idx - jnp.where(tie, idx & 1, 0)
        out_v[pl.ds(i, _L)] = plsc.load_gather(tbl_v, [idx])

    plsc.parallel_loop(0, _HALF, _L, unroll=4)(step)
    cp_o0 = pltpu.async_copy(
        out_v.at[pl.ds(0, _HALF)], out_hbm.at[pl.ds(base, _HALF)], sem_o0
    )
    plsc.parallel_loop(_HALF, _CHUNK, _L, unroll=4)(step)
    cp_o1 = pltpu.async_copy(
        out_v.at[pl.ds(_HALF, _HALF)], out_hbm.at[pl.ds(base + _HALF, _HALF)], sem_o1
    )
    cp_o0.wait()
    cp_o1.wait()


def kernel(t_normalized, betas):
    return _betas_lookup(t_normalized, betas)
